# baseline (speedup 1.0000x reference)
import functools

import jax
import jax.numpy as jnp
from jax import lax
from jax.experimental import pallas as pl
from jax.experimental.pallas import tpu as pltpu
from jax.experimental.pallas import tpu_sc as plsc

_SU, _SL, _TU, _TL = 100.0, 0.0, 500.0, 0.0
_B, _L, _LOC, _D = 16, 50, 1024, 16
_NC, _NS = 2, 16
_IPW = 32
_NACT = (_B * _L) // _IPW
_BPG = 4
_R = _BPG * _L


def _sc_gather_rows(mat2, idx):
    mesh = plsc.VectorSubcoreMesh(core_axis_name="c", subcore_axis_name="s")

    @functools.partial(
        pl.kernel,
        mesh=mesh,
        out_type=jax.ShapeDtypeStruct((_B * _L, _LOC), jnp.float32),
        scratch_types=[
            pltpu.VMEM((_IPW,), jnp.int32),
            pltpu.VMEM((_IPW, _LOC), jnp.float32),
            pltpu.SemaphoreType.DMA,
        ],
    )
    def gather_k(tab_hbm, idx_hbm, out_hbm, idx_v, rows_v, sem):
        wid = lax.axis_index("s") * _NC + lax.axis_index("c")

        @pl.when(wid < _NACT)
        def _():
            base = wid * _IPW
            pltpu.sync_copy(idx_hbm.at[pl.ds(base, _IPW)], idx_v)
            pltpu.async_copy(tab_hbm.at[idx_v], rows_v, sem).wait()
            pltpu.sync_copy(rows_v, out_hbm.at[pl.ds(base, _IPW)])

    return gather_k(mat2, idx)


def _tc_body(tl_s, g_ref, vv_ref, w_ref, out_ref):
    c = pl.program_id(0)
    g = g_ref[...]
    vv = vv_ref[0, 0, :]
    w = w_ref[...]
    sl0, sl1, su1 = w[0:1], w[1:2], w[3:4]
    t0, t1, u0, u1 = w[4:5], w[5:6], w[6:7], w[7:8]
    a0 = sl0 + t0
    a1 = sl1 + t1
    b0 = (u0 - t0) * (1.0 / (_TU - _TL))
    b1 = (u1 - t1) * (1.0 / (_TU - _TL))
    s1 = (su1 - sl1) * (1.0 / (_SU - _SL))
    li = lax.broadcasted_iota(jnp.int32, (_L, _D), 0)
    mc = jnp.concatenate(
        [(li < tl_s[c * _BPG + j]).astype(jnp.float32) for j in range(_BPG)],
        axis=0)
    base = a0 + mc * (a1 - a0) + vv[:, None] * (b0 + mc * (b1 - b0))
    s1l = mc * s1
    out_ref[...] = base[:, :, None] + g[:, None, :] * s1l[:, :, None]


def _tc_expand(g, vec3, wall, tlen):
    grid_spec = pltpu.PrefetchScalarGridSpec(
        num_scalar_prefetch=1,
        grid=(_B // _BPG,),
        in_specs=[
            pl.BlockSpec((_R, _LOC), lambda c, s: (c, 0)),
            pl.BlockSpec((1, 1, _R), lambda c, s: (c, 0, 0)),
            pl.BlockSpec((8, _D), lambda c, s: (0, 0)),
        ],
        out_specs=pl.BlockSpec((_R, _D, _LOC), lambda c, s: (c, 0, 0)),
    )
    return pl.pallas_call(
        _tc_body,
        grid_spec=grid_spec,
        out_shape=jax.ShapeDtypeStruct((_B * _L, _D, _LOC), jnp.float32),
    )(tlen, g, vec3, wall)


def kernel(traj_loc, mat2, vec, traj_len, W_sl, W_su, W_tl, W_tu):
    idx = (traj_loc.astype(jnp.int32) - 1).reshape(_B * _L)
    g = _sc_gather_rows(mat2, idx)
    wall = jnp.concatenate([W_sl, W_su, W_tl, W_tu], axis=0)
    res = _tc_expand(
        g,
        vec.astype(jnp.float32).reshape(_B // _BPG, 1, _R),
        wall,
        traj_len.astype(jnp.int32),
    )
    return res.reshape(_B, _L, _D, _LOC).transpose(0, 1, 3, 2)


# speedup vs baseline: 1.0245x; 1.0245x over previous
import functools

import jax
import jax.numpy as jnp
from jax import lax
from jax.experimental import pallas as pl
from jax.experimental.pallas import tpu as pltpu
from jax.experimental.pallas import tpu_sc as plsc

_SU, _SL, _TU, _TL = 100.0, 0.0, 500.0, 0.0
_B, _L, _LOC, _D = 16, 50, 1024, 16
_NC, _NS = 2, 16
_IPW = 32
_NACT = (_B * _L) // _IPW
_R = 80
_NSTEP = (_B * _L) // _R


def _sc_gather_rows(mat2, idx):
    mesh = plsc.VectorSubcoreMesh(core_axis_name="c", subcore_axis_name="s")

    @functools.partial(
        pl.kernel,
        mesh=mesh,
        out_type=jax.ShapeDtypeStruct((_B * _L, _LOC), jnp.float32),
        scratch_types=[
            pltpu.VMEM((_IPW,), jnp.int32),
            pltpu.VMEM((_IPW, _LOC), jnp.float32),
            pltpu.SemaphoreType.DMA,
        ],
    )
    def gather_k(tab_hbm, idx_hbm, out_hbm, idx_v, rows_v, sem):
        wid = lax.axis_index("s") * _NC + lax.axis_index("c")

        @pl.when(wid < _NACT)
        def _():
            base = wid * _IPW
            pltpu.sync_copy(idx_hbm.at[pl.ds(base, _IPW)], idx_v)
            pltpu.async_copy(tab_hbm.at[idx_v], rows_v, sem).wait()
            pltpu.sync_copy(rows_v, out_hbm.at[pl.ds(base, _IPW)])

    return gather_k(mat2, idx)


def _tc_body(g_ref, vv_ref, tl_ref, w_ref, out_ref):
    c = pl.program_id(0)
    g = g_ref[...]
    vv = vv_ref[0, 0, :]
    tl = tl_ref[0, 0, :]
    w = w_ref[...]
    sl0, sl1, su1 = w[0:1], w[1:2], w[3:4]
    t0, t1, u0, u1 = w[4:5], w[5:6], w[6:7], w[7:8]
    a0 = sl0 + t0
    a1 = sl1 + t1
    b0 = (u0 - t0) * (1.0 / (_TU - _TL))
    b1 = (u1 - t1) * (1.0 / (_TU - _TL))
    s1 = (su1 - sl1) * (1.0 / (_SU - _SL))
    ri = lax.broadcasted_iota(jnp.int32, (_R, _D), 0) + c * _R
    li = jnp.mod(ri, _L)
    mc = (li < tl.astype(jnp.int32)[:, None]).astype(jnp.float32)
    base = a0 + mc * (a1 - a0) + vv[:, None] * (b0 + mc * (b1 - b0))
    s1l = mc * s1
    out_ref[...] = base[:, :, None] + g[:, None, :] * s1l[:, :, None]


def _tc_expand(g, vec3, tl3, wall):
    return pl.pallas_call(
        _tc_body,
        grid=(_NSTEP,),
        in_specs=[
            pl.BlockSpec((_R, _LOC), lambda c: (c, 0)),
            pl.BlockSpec((1, 1, _R), lambda c: (c, 0, 0)),
            pl.BlockSpec((1, 1, _R), lambda c: (c, 0, 0)),
            pl.BlockSpec((8, _D), lambda c: (0, 0)),
        ],
        out_specs=pl.BlockSpec((_R, _D, _LOC), lambda c: (c, 0, 0)),
        out_shape=jax.ShapeDtypeStruct((_B * _L, _D, _LOC), jnp.float32),
    )(g, vec3, tl3, wall)


def kernel(traj_loc, mat2, vec, traj_len, W_sl, W_su, W_tl, W_tu):
    idx = (traj_loc.astype(jnp.int32) - 1).reshape(_B * _L)
    g = _sc_gather_rows(mat2, idx)
    wall = jnp.concatenate([W_sl, W_su, W_tl, W_tu], axis=0)
    tl_row = jnp.repeat(traj_len.astype(jnp.float32), _L)
    res = _tc_expand(
        g,
        vec.astype(jnp.float32).reshape(_NSTEP, 1, _R),
        tl_row.reshape(_NSTEP, 1, _R),
        wall,
    )
    return res.reshape(_B, _L, _D, _LOC).transpose(0, 1, 3, 2)
